# baseline (speedup 1.0000x reference)
import jax
import jax.numpy as jnp
from jax import lax
from jax.experimental import pallas as pl
from jax.experimental.pallas import tpu as pltpu
from jax.experimental.pallas import tpu_sc as plsc

_NUM_TASKS = 1000
_TOTAL_HEADS = 32
_NUM_HEADS = 8
_GROUPS = _TOTAL_HEADS // _NUM_HEADS
_BATCH = 16384

_NC = 2
_NS = 16
_LANES = 16

_ROWS_PER_TILE = 64
_LAST_BASE = _NUM_TASKS - _ROWS_PER_TILE
_B_PER_W = _BATCH // (_NC * _NS)
_BLK = 128
_BLOCKS = _BATCH // _BLK
_BLOCKS_PER_W = _B_PER_W // _BLK


def _body(sc_hbm, tids_hbm, outw_hbm, outi_hbm,
          hl_v, g1_v, g2_v, ti_loc, ti_sh, ti_v,
          tids_v, ow_v, oi_v, dsem):
    s = lax.axis_index("s")
    c = lax.axis_index("c")
    ones16 = jnp.full((_LANES,), 1.0, jnp.float32)
    three16 = jnp.full((_LANES,), 3, jnp.int32)

    wid = c * _NS + s
    base = wid * _B_PER_W
    rbase = jnp.minimum(s * _ROWS_PER_TILE, _LAST_BASE)
    cp_hl = pltpu.async_copy(sc_hbm.at[0, :, pl.ds(rbase, _ROWS_PER_TILE)], hl_v, dsem)
    cp_g1 = pltpu.async_copy(sc_hbm.at[1, :, pl.ds(rbase, _ROWS_PER_TILE)], g1_v, dsem)
    cp_g2 = pltpu.async_copy(sc_hbm.at[2, :, pl.ds(rbase, _ROWS_PER_TILE)], g2_v, dsem)
    cp_ti = pltpu.async_copy(tids_hbm.at[pl.ds(base, _B_PER_W)], tids_v, dsem)
    cp_hl.wait()
    cp_g1.wait()
    cp_g2.wait()
    cp_ti.wait()

    for c4 in range(_ROWS_PER_TILE // _LANES):
        tsl = pl.ds(c4 * _LANES, _LANES)
        packed = jnp.zeros((_LANES,), jnp.int32)
        for h in range(_NUM_HEADS):
            best = hl_v[h, tsl] + g1_v[h, tsl] - g2_v[h, tsl]
            bg = jnp.zeros((_LANES,), jnp.int32)
            for g in range(1, _GROUPS):
                col = g * _NUM_HEADS + h
                v = hl_v[col, tsl] + g1_v[col, tsl] - g2_v[col, tsl]
                m = v > best
                best = jnp.where(m, v, best)
                bg = jnp.where(m, g, bg)
            packed = packed | (bg << (2 * h))
        ti_loc[tsl] = packed

    pltpu.sync_copy(ti_loc, ti_sh.at[pl.ds(rbase, _ROWS_PER_TILE)])
    plsc.subcore_barrier()
    pltpu.sync_copy(ti_sh, ti_v)

    def grp(q, _):
        tv = tids_v[pl.ds(q * _LANES, _LANES)]
        w = plsc.load_gather(ti_v, [tv])
        blk = q >> 3
        off = (q & 7) * _LANES
        for h in range(_NUM_HEADS):
            bg = (w >> (2 * h)) & three16
            oi_v[blk, h, pl.ds(off, _LANES)] = (bg << 3) | h
            ow_v[blk, h, pl.ds(off, _LANES)] = ones16
        return _

    lax.fori_loop(0, _B_PER_W // _LANES, grp, None)

    cp_ow = pltpu.async_copy(ow_v, outw_hbm.at[pl.ds(wid * _BLOCKS_PER_W, _BLOCKS_PER_W)], dsem)
    cp_oi = pltpu.async_copy(oi_v, outi_hbm.at[pl.ds(wid * _BLOCKS_PER_W, _BLOCKS_PER_W)], dsem)
    cp_ow.wait()
    cp_oi.wait()


_sc_call = pl.kernel(
    _body,
    out_type=(
        jax.ShapeDtypeStruct((_BLOCKS, _NUM_HEADS, _BLK), jnp.float32),
        jax.ShapeDtypeStruct((_BLOCKS, _NUM_HEADS, _BLK), jnp.int32),
    ),
    mesh=plsc.VectorSubcoreMesh(core_axis_name="c", subcore_axis_name="s"),
    scratch_types=[
        pltpu.VMEM((_TOTAL_HEADS, _ROWS_PER_TILE), jnp.float32),
        pltpu.VMEM((_TOTAL_HEADS, _ROWS_PER_TILE), jnp.float32),
        pltpu.VMEM((_TOTAL_HEADS, _ROWS_PER_TILE), jnp.float32),
        pltpu.VMEM((_ROWS_PER_TILE,), jnp.int32),
        pltpu.VMEM_SHARED((_NUM_TASKS,), jnp.int32),
        pltpu.VMEM((_NUM_TASKS,), jnp.int32),
        pltpu.VMEM((_B_PER_W,), jnp.int32),
        pltpu.VMEM((_BLOCKS_PER_W, _NUM_HEADS, _BLK), jnp.float32),
        pltpu.VMEM((_BLOCKS_PER_W, _NUM_HEADS, _BLK), jnp.int32),
        pltpu.SemaphoreType.DMA,
    ],
    compiler_params=pltpu.CompilerParams(needs_layout_passes=False,
                                         use_tc_tiling_on_sc=False),
)


def kernel(task_ids, layer_idx, head_logits, gumbels1, gumbels2):
    hl = lax.dynamic_index_in_dim(head_logits, layer_idx, 1, keepdims=False)
    g1 = lax.dynamic_index_in_dim(gumbels1, layer_idx, 1, keepdims=False)
    g2 = lax.dynamic_index_in_dim(gumbels2, layer_idx, 1, keepdims=False)
    stacked = jnp.stack([hl, g1, g2]).transpose(0, 2, 1)
    outw, outi = _sc_call(stacked, task_ids.astype(jnp.int32))
    outw = outw.transpose(0, 2, 1).reshape(_BATCH, _NUM_HEADS)
    outi = outi.transpose(0, 2, 1).reshape(_BATCH, _NUM_HEADS)
    return (outi, outw)


# speedup vs baseline: 1.0029x; 1.0029x over previous
import jax
import jax.numpy as jnp
from jax import lax
from jax.experimental import pallas as pl
from jax.experimental.pallas import tpu as pltpu
from jax.experimental.pallas import tpu_sc as plsc

_NUM_TASKS = 1000
_TOTAL_HEADS = 32
_NUM_HEADS = 8
_GROUPS = _TOTAL_HEADS // _NUM_HEADS
_BATCH = 16384

_NC = 2
_NS = 16
_LANES = 16

_ROWS_PER_TILE = 64
_LAST_BASE = _NUM_TASKS - _ROWS_PER_TILE
_B_PER_W = _BATCH // (_NC * _NS)
_BLK = 128
_BLOCKS = _BATCH // _BLK
_BLOCKS_PER_W = _B_PER_W // _BLK


def _body(sc_hbm, tids_hbm, outw_hbm, outi_hbm,
          hl_v, g1_v, g2_v, ti_loc, ti_sh, ti_v,
          tids_v, ow_v, oi_v, dsem):
    s = lax.axis_index("s")
    c = lax.axis_index("c")
    ones16 = jnp.full((_LANES,), 1.0, jnp.float32)
    three16 = jnp.full((_LANES,), 3, jnp.int32)

    wid = c * _NS + s
    base = wid * _B_PER_W
    rbase = jnp.minimum(s * _ROWS_PER_TILE, _LAST_BASE)
    tt = rbase >> 7
    toff = pl.multiple_of(rbase & (_BLK - 1), 8)
    cp_hl = pltpu.async_copy(sc_hbm.at[0, :, tt, :, pl.ds(toff, _ROWS_PER_TILE)], hl_v, dsem)
    cp_g1 = pltpu.async_copy(sc_hbm.at[1, :, tt, :, pl.ds(toff, _ROWS_PER_TILE)], g1_v, dsem)
    cp_g2 = pltpu.async_copy(sc_hbm.at[2, :, tt, :, pl.ds(toff, _ROWS_PER_TILE)], g2_v, dsem)
    cp_ti = pltpu.async_copy(tids_hbm.at[pl.ds(base, _B_PER_W)], tids_v, dsem)
    cp_hl.wait()
    cp_g1.wait()
    cp_g2.wait()
    cp_ti.wait()

    for c4 in range(_ROWS_PER_TILE // _LANES):
        tsl = pl.ds(c4 * _LANES, _LANES)
        packed = jnp.zeros((_LANES,), jnp.int32)
        for h in range(_NUM_HEADS):
            best = hl_v[0, h, tsl] + g1_v[0, h, tsl] - g2_v[0, h, tsl]
            bg = jnp.zeros((_LANES,), jnp.int32)
            for g in range(1, _GROUPS):
                v = hl_v[g, h, tsl] + g1_v[g, h, tsl] - g2_v[g, h, tsl]
                m = v > best
                best = jnp.where(m, v, best)
                bg = jnp.where(m, g, bg)
            packed = packed | (bg << (2 * h))
        ti_loc[tsl] = packed

    pltpu.sync_copy(ti_loc, ti_sh.at[pl.ds(rbase, _ROWS_PER_TILE)])
    plsc.subcore_barrier()
    pltpu.sync_copy(ti_sh, ti_v)

    def grp(q, _):
        tv = tids_v[pl.ds(q * _LANES, _LANES)]
        w = plsc.load_gather(ti_v, [tv])
        blk = q >> 3
        off = (q & 7) * _LANES
        for h in range(_NUM_HEADS):
            bg = (w >> (2 * h)) & three16
            oi_v[blk, h, pl.ds(off, _LANES)] = (bg << 3) | h
            ow_v[blk, h, pl.ds(off, _LANES)] = ones16
        return _

    lax.fori_loop(0, _B_PER_W // _LANES, grp, None)

    cp_ow = pltpu.async_copy(ow_v, outw_hbm.at[pl.ds(wid * _BLOCKS_PER_W, _BLOCKS_PER_W)], dsem)
    cp_oi = pltpu.async_copy(oi_v, outi_hbm.at[pl.ds(wid * _BLOCKS_PER_W, _BLOCKS_PER_W)], dsem)
    cp_ow.wait()
    cp_oi.wait()


_sc_call = pl.kernel(
    _body,
    out_type=(
        jax.ShapeDtypeStruct((_BLOCKS, _NUM_HEADS, _BLK), jnp.float32),
        jax.ShapeDtypeStruct((_BLOCKS, _NUM_HEADS, _BLK), jnp.int32),
    ),
    mesh=plsc.VectorSubcoreMesh(core_axis_name="c", subcore_axis_name="s"),
    scratch_types=[
        pltpu.VMEM((_GROUPS, _NUM_HEADS, _ROWS_PER_TILE), jnp.float32),
        pltpu.VMEM((_GROUPS, _NUM_HEADS, _ROWS_PER_TILE), jnp.float32),
        pltpu.VMEM((_GROUPS, _NUM_HEADS, _ROWS_PER_TILE), jnp.float32),
        pltpu.VMEM((_ROWS_PER_TILE,), jnp.int32),
        pltpu.VMEM_SHARED((_NUM_TASKS,), jnp.int32),
        pltpu.VMEM((_NUM_TASKS,), jnp.int32),
        pltpu.VMEM((_B_PER_W,), jnp.int32),
        pltpu.VMEM((_BLOCKS_PER_W, _NUM_HEADS, _BLK), jnp.float32),
        pltpu.VMEM((_BLOCKS_PER_W, _NUM_HEADS, _BLK), jnp.int32),
        pltpu.SemaphoreType.DMA,
    ],
    compiler_params=pltpu.CompilerParams(needs_layout_passes=False,
                                         use_tc_tiling_on_sc=False),
)


def kernel(task_ids, layer_idx, head_logits, gumbels1, gumbels2):
    hl = lax.dynamic_index_in_dim(head_logits, layer_idx, 1, keepdims=False)
    g1 = lax.dynamic_index_in_dim(gumbels1, layer_idx, 1, keepdims=False)
    g2 = lax.dynamic_index_in_dim(gumbels2, layer_idx, 1, keepdims=False)
    stacked = jnp.stack([hl, g1, g2]).transpose(0, 2, 1)
    padded = jnp.pad(stacked, ((0, 0), (0, 0), (0, _NUM_HEADS * _BLK - _NUM_TASKS)))
    x5 = padded.reshape(3, _GROUPS, _NUM_HEADS, 8, _BLK).transpose(0, 1, 3, 2, 4)
    outw, outi = _sc_call(x5, task_ids.astype(jnp.int32))
    outw = outw.transpose(0, 2, 1).reshape(_BATCH, _NUM_HEADS)
    outi = outi.transpose(0, 2, 1).reshape(_BATCH, _NUM_HEADS)
    return (outi, outw)
